# baseline (speedup 1.0000x reference)
import functools

import jax
import jax.numpy as jnp
from jax import lax
from jax.experimental import pallas as pl
from jax.experimental.pallas import tpu as pltpu
from jax.experimental.pallas import tpu_sc as plsc

V = 100000
E = 128
D = 128
H = 100
HP = 128
LBL = 104
B = 64
L = 50
NSLOT = 9
NNODE = B * L
NROWS = NSLOT * NNODE


def _ceil_to(x, m):
    return (x + m - 1) // m * m


def _chunk_sizes(total, chunk):
    sizes = [chunk] * (total // chunk)
    if total % chunk:
        sizes.append(total % chunk)
    return sizes


def _sc_gather(idx_pad, table):
    info = plsc.get_sparse_core_info()
    nc, ns = info.num_cores, info.num_subcores
    npad = idx_pad.shape[0]
    r0 = 1112
    r1 = npad // ns - r0
    chunk = 128
    nb = 3
    per_core = {0: (0, r0), 1: (ns * r0, r1)}

    mesh = plsc.VectorSubcoreMesh(core_axis_name="c", subcore_axis_name="s")

    @functools.partial(
        pl.kernel,
        mesh=mesh,
        out_type=jax.ShapeDtypeStruct((npad, E), jnp.float32),
        scratch_types=[
            pltpu.VMEM((max(r0, r1),), jnp.int32),
            pltpu.VMEM((nb, chunk, E), jnp.float32),
        ] + [pltpu.SemaphoreType.DMA] * (2 * nb),
    )
    def gather_kernel(idx_hbm, table_hbm, out_hbm, idx_v, rows_v, *sems):
        gsems, ssems = sems[:nb], sems[nb:]
        sid = lax.axis_index("s")
        cid = lax.axis_index("c")
        for core, (core_base, rows) in per_core.items():
            sizes = _chunk_sizes(rows, chunk)
            n = len(sizes)

            @pl.when(cid == core)
            def _():
                base = core_base + sid * rows
                pltpu.sync_copy(idx_hbm.at[pl.ds(base, rows)], idx_v.at[pl.ds(0, rows)])
                gcp = {}
                for c in range(min(nb, n)):
                    gcp[c] = pltpu.async_copy(
                        table_hbm.at[idx_v.at[pl.ds(c * chunk, sizes[c])]],
                        rows_v.at[c % nb].at[pl.ds(0, sizes[c])], gsems[c % nb])
                scp = {}
                for c in range(n):
                    b = c % nb
                    gcp[c].wait()
                    scp[c] = pltpu.async_copy(
                        rows_v.at[b].at[pl.ds(0, sizes[c])],
                        out_hbm.at[pl.ds(base + c * chunk, sizes[c])], ssems[b])
                    if c + nb < n:
                        scp[c].wait()
                        gcp[c + nb] = pltpu.async_copy(
                            table_hbm.at[idx_v.at[pl.ds((c + nb) * chunk, sizes[c + nb])]],
                            rows_v.at[b].at[pl.ds(0, sizes[c + nb])], gsems[b])
                for c in range(max(n - nb, 0), n):
                    scp[c].wait()

    return gather_kernel(idx_pad, table)


def _dot_t(x, w):
    return lax.dot_general(x, w, (((1,), (1,)), ((), ())),
                           preferred_element_type=jnp.float32)


def _tc_body(g_ref, wc_ref, bc_ref,
             wihf_ref, whhf_ref, bihf_ref, bhhf_ref,
             wihb_ref, whhb_ref, bihb_ref, bhhb_ref,
             wout_ref, bout_ref,
             feat_ref, out_ref,
             gif_ref, gib_ref):
    bc = bc_ref[...]
    total = None
    mx = None
    for s in range(NSLOT):
        g = g_ref[s * NNODE:(s + 1) * NNODE, :]
        enc = _dot_t(g, wc_ref[...]) + bc
        if s == 0:
            total = enc
        else:
            total = total + enc
            mx = enc if s == 1 else jnp.maximum(mx, enc)
    encodes = jnp.maximum(total, mx)

    gif = _dot_t(encodes, wihf_ref[...]) + bihf_ref[...]
    gib = _dot_t(encodes, wihb_ref[...]) + bihb_ref[...]
    gif_ref[...] = gif.reshape(L, B, 3 * HP)
    gib_ref[...] = gib.reshape(L, B, 3 * HP)

    whhf = whhf_ref[...]
    whhb = whhb_ref[...]
    bhhf = bhhf_ref[...]
    bhhb = bhhb_ref[...]

    def _sigmoid(x):
        return 0.5 * jnp.tanh(0.5 * x) + 0.5

    def _gru_dir(g, gh, h):
        r = _sigmoid(g[:, :HP] + gh[:, :HP])
        z = _sigmoid(g[:, HP:2 * HP] + gh[:, HP:2 * HP])
        n = jnp.tanh(g[:, 2 * HP:] + r * gh[:, 2 * HP:])
        return n + z * (h - n)

    def step(t, carry):
        hf, hb, mf, mb = carry
        gf = gif_ref[t]
        gb = gib_ref[L - 1 - t]
        ghf = _dot_t(hf, whhf) + bhhf
        ghb = _dot_t(hb, whhb) + bhhb
        hf = _gru_dir(gf, ghf, hf)
        hb = _gru_dir(gb, ghb, hb)
        return hf, hb, jnp.maximum(mf, hf), jnp.maximum(mb, hb)

    z = jnp.zeros((B, HP), jnp.float32)
    neg = jnp.full((B, HP), -2.0, jnp.float32)
    _, _, mf, mb = lax.fori_loop(0, L, step, (z, z, neg, neg))
    feats = jnp.concatenate([mf, mb], axis=1)
    feat_ref[...] = feats
    out_ref[...] = _dot_t(feats, wout_ref[...]) + bout_ref[...]


def _pad_gates(w, n, cols_to=None):
    w3 = w.reshape(3, n, w.shape[1])
    w3 = jnp.pad(w3, ((0, 0), (0, HP - n), (0, 0)))
    if cols_to is not None:
        w3 = jnp.pad(w3, ((0, 0), (0, 0), (0, cols_to - w3.shape[2])))
    return w3.reshape(3 * HP, -1)


def _pad_gate_bias(b):
    return jnp.pad(b.reshape(3, H), ((0, 0), (0, HP - H))).reshape(1, 3 * HP)


def kernel(x, emb, W_c_w, W_c_b, Wih_f, Whh_f, bih_f, bhh_f,
           Wih_b, Whh_b, bih_b, bhh_b, W_out, b_out):
    idx = jnp.transpose(x, (2, 1, 0)).reshape(-1).astype(jnp.int32)
    npad = _ceil_to(NROWS, 32 * 8)
    idx_pad = jnp.pad(idx, (0, npad - NROWS))

    gathered = _sc_gather(idx_pad, emb)

    wihf = _pad_gates(Wih_f, H)
    wihb = _pad_gates(Wih_b, H)
    whhf = _pad_gates(Whh_f, H, cols_to=HP)
    whhb = _pad_gates(Whh_b, H, cols_to=HP)
    bihf = _pad_gate_bias(bih_f)
    bihb = _pad_gate_bias(bih_b)
    bhhf = _pad_gate_bias(bhh_f)
    bhhb = _pad_gate_bias(bhh_b)
    wout = jnp.pad(W_out.reshape(LBL, 2, H),
                   ((0, 0), (0, 0), (0, HP - H))).reshape(LBL, 2 * HP)
    bout = b_out.reshape(1, LBL)
    bc = W_c_b.reshape(1, D)

    feats_pad, outputs = pl.pallas_call(
        _tc_body,
        out_shape=(
            jax.ShapeDtypeStruct((B, 2 * HP), jnp.float32),
            jax.ShapeDtypeStruct((B, LBL), jnp.float32),
        ),
        scratch_shapes=[
            pltpu.VMEM((L, B, 3 * HP), jnp.float32),
            pltpu.VMEM((L, B, 3 * HP), jnp.float32),
        ],
    )(gathered, W_c_w, bc,
      wihf, whhf, bihf, bhhf,
      wihb, whhb, bihb, bhhb,
      wout, bout)

    features = jnp.concatenate(
        [feats_pad[:, :H], feats_pad[:, HP:HP + H]], axis=1)
    return (features, outputs)


# speedup vs baseline: 1.0130x; 1.0130x over previous
import functools

import jax
import jax.numpy as jnp
from jax import lax
from jax.experimental import pallas as pl
from jax.experimental.pallas import tpu as pltpu
from jax.experimental.pallas import tpu_sc as plsc

V = 100000
E = 128
D = 128
H = 100
HP = 128
LBL = 104
B = 64
L = 50
NSLOT = 9
NNODE = B * L
NROWS = NSLOT * NNODE


def _ceil_to(x, m):
    return (x + m - 1) // m * m


def _chunk_sizes(total, chunk):
    sizes = [chunk] * (total // chunk)
    if total % chunk:
        sizes.append(total % chunk)
    return sizes


def _sc_gather(idx_pad, table):
    info = plsc.get_sparse_core_info()
    nc, ns = info.num_cores, info.num_subcores
    npad = idx_pad.shape[0]
    r0 = 1136
    r1 = npad // ns - r0
    chunk = 128
    nb = r1 // chunk

    mesh = plsc.VectorSubcoreMesh(core_axis_name="c", subcore_axis_name="s")

    @functools.partial(
        pl.kernel,
        mesh=mesh,
        out_type=jax.ShapeDtypeStruct((npad, E), jnp.float32),
        scratch_types=[
            pltpu.VMEM((r0,), jnp.int32),
            pltpu.VMEM((r1, E), jnp.float32),
        ] + [pltpu.SemaphoreType.DMA] * (max(nb, -(-r1 // chunk)) + nb + 1),
    )
    def gather_kernel(idx_hbm, table_hbm, out_hbm, idx_v, buf_v, *sems):
        ng = max(nb, -(-r1 // chunk))
        gsems, ssems = sems[:ng], sems[ng:ng + nb]
        st_sem = sems[ng + nb]
        sid = lax.axis_index("s")
        cid = lax.axis_index("c")

        sizes = _chunk_sizes(r0, chunk)
        n = len(sizes)

        @pl.when(cid == 0)
        def _():
            base = sid * r0
            pltpu.sync_copy(idx_hbm.at[pl.ds(base, r0)], idx_v)
            gcp = {}
            scp = {}
            for c in range(min(nb, n)):
                gcp[c] = pltpu.async_copy(
                    table_hbm.at[idx_v.at[pl.ds(c * chunk, sizes[c])]],
                    buf_v.at[pl.ds((c % nb) * chunk, sizes[c])], gsems[c % nb])
            for c in range(n):
                b = c % nb
                j = c - 1 + nb
                if c > 0 and j < n:
                    scp[c - 1].wait()
                    gcp[j] = pltpu.async_copy(
                        table_hbm.at[idx_v.at[pl.ds(j * chunk, sizes[j])]],
                        buf_v.at[pl.ds((j % nb) * chunk, sizes[j])], gsems[j % nb])
                gcp[c].wait()
                scp[c] = pltpu.async_copy(
                    buf_v.at[pl.ds(b * chunk, sizes[c])],
                    out_hbm.at[pl.ds(base + c * chunk, sizes[c])], ssems[b])
            for c in range(max(n - nb, 0), n):
                scp[c].wait()

        sizes1 = _chunk_sizes(r1, chunk)
        n1 = len(sizes1)

        @pl.when(cid == 1)
        def _():
            base = ns * r0 + sid * r1
            pltpu.sync_copy(idx_hbm.at[pl.ds(base, r1)], idx_v.at[pl.ds(0, r1)])
            gcp = {}
            for c in range(n1):
                gcp[c] = pltpu.async_copy(
                    table_hbm.at[idx_v.at[pl.ds(c * chunk, sizes1[c])]],
                    buf_v.at[pl.ds(c * chunk, sizes1[c])], gsems[c])
            scp = {}
            for c in range(n1):
                gcp[c].wait()
                scp[c] = pltpu.async_copy(
                    buf_v.at[pl.ds(c * chunk, sizes1[c])],
                    out_hbm.at[pl.ds(base + c * chunk, sizes1[c])], st_sem)
            for c in range(n1):
                scp[c].wait()

    return gather_kernel(idx_pad, table)


def _dot_t(x, w):
    return lax.dot_general(x, w, (((1,), (1,)), ((), ())),
                           preferred_element_type=jnp.float32)


def _tc_body(g_ref, wc_ref, bc_ref,
             wihf_ref, whhf_ref, bihf_ref, bhhf_ref,
             wihb_ref, whhb_ref, bihb_ref, bhhb_ref,
             wout_ref, bout_ref,
             feat_ref, out_ref,
             gif_ref, gib_ref):
    bc = bc_ref[...]
    total = None
    mx = None
    for s in range(NSLOT):
        g = g_ref[s * NNODE:(s + 1) * NNODE, :]
        enc = _dot_t(g, wc_ref[...]) + bc
        if s == 0:
            total = enc
        else:
            total = total + enc
            mx = enc if s == 1 else jnp.maximum(mx, enc)
    encodes = jnp.maximum(total, mx)

    enc_bf = encodes.astype(jnp.bfloat16)
    gif = _dot_t(enc_bf, wihf_ref[...]) + bihf_ref[...]
    gib = _dot_t(enc_bf, wihb_ref[...]) + bihb_ref[...]
    gif_ref[...] = gif.reshape(L, B, 3 * HP)
    gib_ref[...] = gib.reshape(L, B, 3 * HP)

    whhf = whhf_ref[...]
    whhb = whhb_ref[...]
    bhhf = bhhf_ref[...]
    bhhb = bhhb_ref[...]

    def _sigmoid(x):
        return 0.5 * jnp.tanh(0.5 * x) + 0.5

    def _gru_dir(g, gh, h):
        r = _sigmoid(g[:, :HP] + gh[:, :HP])
        z = _sigmoid(g[:, HP:2 * HP] + gh[:, HP:2 * HP])
        n = jnp.tanh(g[:, 2 * HP:] + r * gh[:, 2 * HP:])
        return n + z * (h - n)

    def step(t, carry):
        hf, hb, mf, mb = carry
        gf = gif_ref[t]
        gb = gib_ref[L - 1 - t]
        ghf = _dot_t(hf.astype(jnp.bfloat16), whhf) + bhhf
        ghb = _dot_t(hb.astype(jnp.bfloat16), whhb) + bhhb
        hf = _gru_dir(gf, ghf, hf)
        hb = _gru_dir(gb, ghb, hb)
        return hf, hb, jnp.maximum(mf, hf), jnp.maximum(mb, hb)

    z = jnp.zeros((B, HP), jnp.float32)
    neg = jnp.full((B, HP), -2.0, jnp.float32)
    _, _, mf, mb = lax.fori_loop(0, L, step, (z, z, neg, neg))
    feats = jnp.concatenate([mf, mb], axis=1)
    feat_ref[...] = feats
    out_ref[...] = _dot_t(feats, wout_ref[...]) + bout_ref[...]


def _pad_gates(w, n, cols_to=None):
    w3 = w.reshape(3, n, w.shape[1])
    w3 = jnp.pad(w3, ((0, 0), (0, HP - n), (0, 0)))
    if cols_to is not None:
        w3 = jnp.pad(w3, ((0, 0), (0, 0), (0, cols_to - w3.shape[2])))
    return w3.reshape(3 * HP, -1)


def _pad_gate_bias(b):
    return jnp.pad(b.reshape(3, H), ((0, 0), (0, HP - H))).reshape(1, 3 * HP)


def kernel(x, emb, W_c_w, W_c_b, Wih_f, Whh_f, bih_f, bhh_f,
           Wih_b, Whh_b, bih_b, bhh_b, W_out, b_out):
    idx = jnp.transpose(x, (2, 1, 0)).reshape(-1).astype(jnp.int32)
    npad = _ceil_to(NROWS, 32 * 8)
    idx_pad = jnp.pad(idx, (0, npad - NROWS))

    gathered = _sc_gather(idx_pad, emb)

    wihf = _pad_gates(Wih_f, H).astype(jnp.bfloat16)
    wihb = _pad_gates(Wih_b, H).astype(jnp.bfloat16)
    whhf = _pad_gates(Whh_f, H, cols_to=HP).astype(jnp.bfloat16)
    whhb = _pad_gates(Whh_b, H, cols_to=HP).astype(jnp.bfloat16)
    bihf = _pad_gate_bias(bih_f)
    bihb = _pad_gate_bias(bih_b)
    bhhf = _pad_gate_bias(bhh_f)
    bhhb = _pad_gate_bias(bhh_b)
    wout = jnp.pad(W_out.reshape(LBL, 2, H),
                   ((0, 0), (0, 0), (0, HP - H))).reshape(LBL, 2 * HP)
    bout = b_out.reshape(1, LBL)
    bc = W_c_b.reshape(1, D)

    feats_pad, outputs = pl.pallas_call(
        _tc_body,
        out_shape=(
            jax.ShapeDtypeStruct((B, 2 * HP), jnp.float32),
            jax.ShapeDtypeStruct((B, LBL), jnp.float32),
        ),
        scratch_shapes=[
            pltpu.VMEM((L, B, 3 * HP), jnp.float32),
            pltpu.VMEM((L, B, 3 * HP), jnp.float32),
        ],
    )(gathered, W_c_w, bc,
      wihf, whhf, bihf, bhhf,
      wihb, whhb, bihb, bhhb,
      wout, bout)

    features = jnp.concatenate(
        [feats_pad[:, :H], feats_pad[:, HP:HP + H]], axis=1)
    return (features, outputs)


# speedup vs baseline: 1.0132x; 1.0002x over previous
import functools

import jax
import jax.numpy as jnp
from jax import lax
from jax.experimental import pallas as pl
from jax.experimental.pallas import tpu as pltpu
from jax.experimental.pallas import tpu_sc as plsc

V = 100000
E = 128
D = 128
H = 100
HP = 128
LBL = 104
B = 64
L = 50
NSLOT = 9
NNODE = B * L
NROWS = NSLOT * NNODE


def _ceil_to(x, m):
    return (x + m - 1) // m * m


def _chunk_sizes(total, chunk):
    sizes = [chunk] * (total // chunk)
    if total % chunk:
        sizes.append(total % chunk)
    return sizes


def _sc_gather(idx_pad, table):
    info = plsc.get_sparse_core_info()
    nc, ns = info.num_cores, info.num_subcores
    npad = idx_pad.shape[0]
    r0 = 1488
    r1 = npad // ns - r0
    chunk = 128
    bufrows = max(r1, 5 * chunk)
    nb = bufrows // chunk

    mesh = plsc.VectorSubcoreMesh(core_axis_name="c", subcore_axis_name="s")

    @functools.partial(
        pl.kernel,
        mesh=mesh,
        out_type=jax.ShapeDtypeStruct((npad, E), jnp.float32),
        scratch_types=[
            pltpu.VMEM((r0,), jnp.int32),
            pltpu.VMEM((bufrows, E), jnp.float32),
        ] + [pltpu.SemaphoreType.DMA] * (max(nb, -(-r1 // chunk)) + nb + 1),
    )
    def gather_kernel(idx_hbm, table_hbm, out_hbm, idx_v, buf_v, *sems):
        ng = max(nb, -(-r1 // chunk))
        gsems, ssems = sems[:ng], sems[ng:ng + nb]
        st_sem = sems[ng + nb]
        sid = lax.axis_index("s")
        cid = lax.axis_index("c")

        sizes = _chunk_sizes(r0, chunk)
        n = len(sizes)

        @pl.when(cid == 0)
        def _():
            base = sid * r0
            pltpu.sync_copy(idx_hbm.at[pl.ds(base, r0)], idx_v)
            gcp = {}
            scp = {}
            for c in range(min(nb, n)):
                gcp[c] = pltpu.async_copy(
                    table_hbm.at[idx_v.at[pl.ds(c * chunk, sizes[c])]],
                    buf_v.at[pl.ds((c % nb) * chunk, sizes[c])], gsems[c % nb])
            for c in range(n):
                b = c % nb
                j = c - 1 + nb
                if c > 0 and j < n:
                    scp[c - 1].wait()
                    gcp[j] = pltpu.async_copy(
                        table_hbm.at[idx_v.at[pl.ds(j * chunk, sizes[j])]],
                        buf_v.at[pl.ds((j % nb) * chunk, sizes[j])], gsems[j % nb])
                gcp[c].wait()
                scp[c] = pltpu.async_copy(
                    buf_v.at[pl.ds(b * chunk, sizes[c])],
                    out_hbm.at[pl.ds(base + c * chunk, sizes[c])], ssems[b])
            for c in range(max(n - nb, 0), n):
                scp[c].wait()

        sizes1 = _chunk_sizes(r1, chunk)
        n1 = len(sizes1)

        @pl.when(cid == 1)
        def _():
            base = ns * r0 + sid * r1
            pltpu.sync_copy(idx_hbm.at[pl.ds(base, r1)], idx_v.at[pl.ds(0, r1)])
            gcp = {}
            for c in range(n1):
                gcp[c] = pltpu.async_copy(
                    table_hbm.at[idx_v.at[pl.ds(c * chunk, sizes1[c])]],
                    buf_v.at[pl.ds(c * chunk, sizes1[c])], gsems[c])
            scp = {}
            for c in range(n1):
                gcp[c].wait()
                scp[c] = pltpu.async_copy(
                    buf_v.at[pl.ds(c * chunk, sizes1[c])],
                    out_hbm.at[pl.ds(base + c * chunk, sizes1[c])], st_sem)
            for c in range(n1):
                scp[c].wait()

    return gather_kernel(idx_pad, table)


def _dot_t(x, w):
    return lax.dot_general(x, w, (((1,), (1,)), ((), ())),
                           preferred_element_type=jnp.float32)


def _tc_body(g_ref, wc_ref, bc_ref,
             wihf_ref, whhf_ref, bihf_ref, bhhf_ref,
             wihb_ref, whhb_ref, bihb_ref, bhhb_ref,
             wout_ref, bout_ref,
             feat_ref, out_ref,
             gif_ref, gib_ref):
    bc = bc_ref[...]
    total = None
    mx = None
    for s in range(NSLOT):
        g = g_ref[s * NNODE:(s + 1) * NNODE, :]
        enc = _dot_t(g, wc_ref[...]) + bc
        if s == 0:
            total = enc
        else:
            total = total + enc
            mx = enc if s == 1 else jnp.maximum(mx, enc)
    encodes = jnp.maximum(total, mx)

    enc_bf = encodes.astype(jnp.bfloat16)
    gif = _dot_t(enc_bf, wihf_ref[...]) + bihf_ref[...]
    gib = _dot_t(enc_bf, wihb_ref[...]) + bihb_ref[...]
    gif_ref[...] = gif.reshape(L, B, 3 * HP)
    gib_ref[...] = gib.reshape(L, B, 3 * HP)

    whhf = whhf_ref[...]
    whhb = whhb_ref[...]
    bhhf = bhhf_ref[...]
    bhhb = bhhb_ref[...]

    def _sigmoid(x):
        return 0.5 * jnp.tanh(0.5 * x) + 0.5

    def _gru_dir(g, gh, bhn, h):
        r = _sigmoid(g[:, :HP] + gh[:, :HP])
        z = _sigmoid(g[:, HP:2 * HP] + gh[:, HP:2 * HP])
        n = jnp.tanh(g[:, 2 * HP:] + r * (gh[:, 2 * HP:] + bhn))
        return n + z * (h - n)

    def step(t, carry):
        hf, hb, mf, mb = carry
        gf = gif_ref[t]
        ghf = _dot_t(hf.astype(jnp.bfloat16), whhf)
        hf = _gru_dir(gf, ghf, bhhf, hf)
        mf = jnp.maximum(mf, hf)
        gb = gib_ref[L - 1 - t]
        ghb = _dot_t(hb.astype(jnp.bfloat16), whhb)
        hb = _gru_dir(gb, ghb, bhhb, hb)
        mb = jnp.maximum(mb, hb)
        return hf, hb, mf, mb

    z = jnp.zeros((B, HP), jnp.float32)
    neg = jnp.full((B, HP), -2.0, jnp.float32)
    _, _, mf, mb = lax.fori_loop(0, L, step, (z, z, neg, neg))
    feats = jnp.concatenate([mf, mb], axis=1)
    feat_ref[...] = feats
    out_ref[...] = _dot_t(feats, wout_ref[...]) + bout_ref[...]


def _pad_gates(w, n, cols_to=None):
    w3 = w.reshape(3, n, w.shape[1])
    w3 = jnp.pad(w3, ((0, 0), (0, HP - n), (0, 0)))
    if cols_to is not None:
        w3 = jnp.pad(w3, ((0, 0), (0, 0), (0, cols_to - w3.shape[2])))
    return w3.reshape(3 * HP, -1)


def _pad_gate_bias(b):
    return jnp.pad(b.reshape(-1, H), ((0, 0), (0, HP - H))).reshape(1, -1)


def _fold_bias(bih, bhh):
    b3 = bih.reshape(3, H)
    h3 = bhh.reshape(3, H)
    folded = jnp.concatenate([b3[:2] + h3[:2], b3[2:]], axis=0)
    return _pad_gate_bias(folded.reshape(-1)), _pad_gate_bias(h3[2])


def kernel(x, emb, W_c_w, W_c_b, Wih_f, Whh_f, bih_f, bhh_f,
           Wih_b, Whh_b, bih_b, bhh_b, W_out, b_out):
    idx = jnp.transpose(x, (2, 1, 0)).reshape(-1).astype(jnp.int32)
    npad = _ceil_to(NROWS, 32 * 8)
    idx_pad = jnp.pad(idx, (0, npad - NROWS))

    gathered = _sc_gather(idx_pad, emb)

    wihf = _pad_gates(Wih_f, H).astype(jnp.bfloat16)
    wihb = _pad_gates(Wih_b, H).astype(jnp.bfloat16)
    whhf = _pad_gates(Whh_f, H, cols_to=HP).astype(jnp.bfloat16)
    whhb = _pad_gates(Whh_b, H, cols_to=HP).astype(jnp.bfloat16)
    bihf, bhhf = _fold_bias(bih_f, bhh_f)
    bihb, bhhb = _fold_bias(bih_b, bhh_b)
    wout = jnp.pad(W_out.reshape(LBL, 2, H),
                   ((0, 0), (0, 0), (0, HP - H))).reshape(LBL, 2 * HP)
    bout = b_out.reshape(1, LBL)
    bc = W_c_b.reshape(1, D)

    feats_pad, outputs = pl.pallas_call(
        _tc_body,
        out_shape=(
            jax.ShapeDtypeStruct((B, 2 * HP), jnp.float32),
            jax.ShapeDtypeStruct((B, LBL), jnp.float32),
        ),
        scratch_shapes=[
            pltpu.VMEM((L, B, 3 * HP), jnp.float32),
            pltpu.VMEM((L, B, 3 * HP), jnp.float32),
        ],
    )(gathered, W_c_w, bc,
      wihf, whhf, bihf, bhhf,
      wihb, whhb, bihb, bhhb,
      wout, bout)

    features = jnp.concatenate(
        [feats_pad[:, :H], feats_pad[:, HP:HP + H]], axis=1)
    return (features, outputs)


# speedup vs baseline: 1.0473x; 1.0337x over previous
import functools

import jax
import jax.numpy as jnp
from jax import lax
from jax.experimental import pallas as pl
from jax.experimental.pallas import tpu as pltpu
from jax.experimental.pallas import tpu_sc as plsc

V = 100000
E = 128
D = 128
H = 100
HP = 128
LBL = 104
B = 64
L = 50
NSLOT = 9
NNODE = B * L
NROWS = NSLOT * NNODE


def _ceil_to(x, m):
    return (x + m - 1) // m * m


def _chunk_sizes(total, chunk):
    sizes = [chunk] * (total // chunk)
    if total % chunk:
        sizes.append(total % chunk)
    return sizes


def _sc_gather(idx_pad, table):
    info = plsc.get_sparse_core_info()
    nc, ns = info.num_cores, info.num_subcores
    npad = idx_pad.shape[0]
    r0 = 1200
    r1 = npad // ns - r0
    chunk = 128
    bufrows = max(r1, 5 * chunk)
    nb = bufrows // chunk

    mesh = plsc.VectorSubcoreMesh(core_axis_name="c", subcore_axis_name="s")

    @functools.partial(
        pl.kernel,
        mesh=mesh,
        out_type=jax.ShapeDtypeStruct((npad, E), jnp.float32),
        scratch_types=[
            pltpu.VMEM((r0,), jnp.int32),
            pltpu.VMEM((bufrows, E), jnp.float32),
        ] + [pltpu.SemaphoreType.DMA] * (max(nb, -(-r1 // chunk)) + nb + 1),
    )
    def gather_kernel(idx_hbm, table_hbm, out_hbm, idx_v, buf_v, *sems):
        ng = max(nb, -(-r1 // chunk))
        gsems, ssems = sems[:ng], sems[ng:ng + nb]
        st_sem = sems[ng + nb]
        sid = lax.axis_index("s")
        cid = lax.axis_index("c")

        sizes = _chunk_sizes(r0, chunk)
        n = len(sizes)

        @pl.when(cid == 0)
        def _():
            base = sid * r0
            pltpu.sync_copy(idx_hbm.at[pl.ds(base, r0)], idx_v)
            gcp = {}
            scp = {}
            for c in range(min(nb, n)):
                gcp[c] = pltpu.async_copy(
                    table_hbm.at[idx_v.at[pl.ds(c * chunk, sizes[c])]],
                    buf_v.at[pl.ds((c % nb) * chunk, sizes[c])], gsems[c % nb])
            for c in range(n):
                b = c % nb
                j = c - 1 + nb
                if c > 0 and j < n:
                    scp[c - 1].wait()
                    gcp[j] = pltpu.async_copy(
                        table_hbm.at[idx_v.at[pl.ds(j * chunk, sizes[j])]],
                        buf_v.at[pl.ds((j % nb) * chunk, sizes[j])], gsems[j % nb])
                gcp[c].wait()
                scp[c] = pltpu.async_copy(
                    buf_v.at[pl.ds(b * chunk, sizes[c])],
                    out_hbm.at[pl.ds(base + c * chunk, sizes[c])], ssems[b])
            for c in range(max(n - nb, 0), n):
                scp[c].wait()

        sizes1 = _chunk_sizes(r1, chunk)
        n1 = len(sizes1)

        @pl.when(cid == 1)
        def _():
            base = ns * r0 + sid * r1
            pltpu.sync_copy(idx_hbm.at[pl.ds(base, r1)], idx_v.at[pl.ds(0, r1)])
            gcp = {}
            for c in range(n1):
                gcp[c] = pltpu.async_copy(
                    table_hbm.at[idx_v.at[pl.ds(c * chunk, sizes1[c])]],
                    buf_v.at[pl.ds(c * chunk, sizes1[c])], gsems[c])
            scp = {}
            for c in range(n1):
                gcp[c].wait()
                scp[c] = pltpu.async_copy(
                    buf_v.at[pl.ds(c * chunk, sizes1[c])],
                    out_hbm.at[pl.ds(base + c * chunk, sizes1[c])], st_sem)
            for c in range(n1):
                scp[c].wait()

    return gather_kernel(idx_pad, table)


def _dot_t(x, w):
    return lax.dot_general(x, w, (((1,), (1,)), ((), ())),
                           preferred_element_type=jnp.float32)


def _tc_body(g_ref, wc_ref, bc_ref,
             wihf_ref, whhf_ref, bihf_ref, bhhf_ref,
             wihb_ref, whhb_ref, bihb_ref, bhhb_ref,
             wout_ref, bout_ref,
             feat_ref, out_ref,
             gif_ref, gib_ref, ysf_ref, ysb_ref):
    bc = bc_ref[...]
    wc = wc_ref[...]
    total = None
    mx = None
    for s in range(NSLOT):
        g = g_ref[s * NNODE:(s + 1) * NNODE, :].astype(jnp.bfloat16)
        enc = _dot_t(g, wc) + bc
        if s == 0:
            total = enc
        else:
            total = total + enc
            mx = enc if s == 1 else jnp.maximum(mx, enc)
    encodes = jnp.maximum(total, mx)

    enc_bf = encodes.astype(jnp.bfloat16)
    gif = _dot_t(enc_bf, wihf_ref[...]) + bihf_ref[...]
    gib = _dot_t(enc_bf, wihb_ref[...]) + bihb_ref[...]
    gif_ref[...] = gif.reshape(L, B, 3 * HP)
    gib_ref[...] = gib.reshape(L, B, 3 * HP)

    whhf = whhf_ref[...]
    whhb = whhb_ref[...]
    bhhf = bhhf_ref[...]
    bhhb = bhhb_ref[...]

    def _sigmoid(x):
        return 0.5 * jnp.tanh(0.5 * x) + 0.5

    def _gru_dir(g, gh, bhn, h):
        r = _sigmoid(g[:, :HP] + gh[:, :HP])
        z = _sigmoid(g[:, HP:2 * HP] + gh[:, HP:2 * HP])
        n = jnp.tanh(g[:, 2 * HP:] + r * (gh[:, 2 * HP:] + bhn))
        return n + z * (h - n)

    def _one(t, hf, hb):
        gf = gif_ref[t]
        gb = gib_ref[L - 1 - t]
        ghf = _dot_t(hf.astype(jnp.bfloat16), whhf)
        ghb = _dot_t(hb.astype(jnp.bfloat16), whhb)
        hf = _gru_dir(gf, ghf, bhhf, hf)
        hb = _gru_dir(gb, ghb, bhhb, hb)
        ysf_ref[t] = hf
        ysb_ref[t] = hb
        return hf, hb

    def step(i, carry):
        hf, hb = carry
        hf, hb = _one(2 * i, hf, hb)
        hf, hb = _one(2 * i + 1, hf, hb)
        return hf, hb

    z = jnp.zeros((B, HP), jnp.float32)
    lax.fori_loop(0, L // 2, step, (z, z))
    mf = ysf_ref[0]
    mb = ysb_ref[0]
    for i in range(1, L):
        mf = jnp.maximum(mf, ysf_ref[i])
        mb = jnp.maximum(mb, ysb_ref[i])
    feats = jnp.concatenate([mf, mb], axis=1)
    feat_ref[...] = feats
    out_ref[...] = _dot_t(feats, wout_ref[...]) + bout_ref[...]


def _pad_gates(w, n, cols_to=None):
    w3 = w.reshape(3, n, w.shape[1])
    w3 = jnp.pad(w3, ((0, 0), (0, HP - n), (0, 0)))
    if cols_to is not None:
        w3 = jnp.pad(w3, ((0, 0), (0, 0), (0, cols_to - w3.shape[2])))
    return w3.reshape(3 * HP, -1)


def _pad_gate_bias(b):
    return jnp.pad(b.reshape(-1, H), ((0, 0), (0, HP - H))).reshape(1, -1)


def _fold_bias(bih, bhh):
    b3 = bih.reshape(3, H)
    h3 = bhh.reshape(3, H)
    folded = jnp.concatenate([b3[:2] + h3[:2], b3[2:]], axis=0)
    return _pad_gate_bias(folded.reshape(-1)), _pad_gate_bias(h3[2])


def kernel(x, emb, W_c_w, W_c_b, Wih_f, Whh_f, bih_f, bhh_f,
           Wih_b, Whh_b, bih_b, bhh_b, W_out, b_out):
    idx = jnp.transpose(x, (2, 1, 0)).reshape(-1).astype(jnp.int32)
    npad = _ceil_to(NROWS, 32 * 8)
    idx_pad = jnp.pad(idx, (0, npad - NROWS))

    gathered = _sc_gather(idx_pad, emb)

    wihf = _pad_gates(Wih_f, H).astype(jnp.bfloat16)
    wihb = _pad_gates(Wih_b, H).astype(jnp.bfloat16)
    whhf = _pad_gates(Whh_f, H, cols_to=HP).astype(jnp.bfloat16)
    whhb = _pad_gates(Whh_b, H, cols_to=HP).astype(jnp.bfloat16)
    bihf, bhhf = _fold_bias(bih_f, bhh_f)
    bihb, bhhb = _fold_bias(bih_b, bhh_b)
    wout = jnp.pad(W_out.reshape(LBL, 2, H),
                   ((0, 0), (0, 0), (0, HP - H))).reshape(LBL, 2 * HP)
    bout = b_out.reshape(1, LBL)
    bc = W_c_b.reshape(1, D)

    feats_pad, outputs = pl.pallas_call(
        _tc_body,
        out_shape=(
            jax.ShapeDtypeStruct((B, 2 * HP), jnp.float32),
            jax.ShapeDtypeStruct((B, LBL), jnp.float32),
        ),
        scratch_shapes=[
            pltpu.VMEM((L, B, 3 * HP), jnp.float32),
            pltpu.VMEM((L, B, 3 * HP), jnp.float32),
            pltpu.VMEM((L, B, HP), jnp.float32),
            pltpu.VMEM((L, B, HP), jnp.float32),
        ],
    )(gathered, W_c_w.astype(jnp.bfloat16), bc,
      wihf, whhf, bihf, bhhf,
      wihb, whhb, bihb, bhhb,
      wout, bout)

    features = jnp.concatenate(
        [feats_pad[:, :H], feats_pad[:, HP:HP + H]], axis=1)
    return (features, outputs)
